# baseline (speedup 1.0000x reference)
import functools

import jax
import jax.numpy as jnp
from jax import lax
from jax.experimental import pallas as pl
from jax.experimental.pallas import tpu as pltpu
from jax.experimental.pallas import tpu_sc as plsc

_B = 16384
_EMB = 16
_HID = 32
_NCTX = 8
_NC = 2
_NS = 16
_NW = _NC * _NS
_BPW = _B // _NW
_CH = 256
_NCHUNK = _BPW // _CH
_ROWW = _EMB + _EMB + _NCTX
_L = 16
_NWTS = _ROWW * _HID + _HID + _HID + 16


def _splat(v, k):
    return lax.gather(
        v, jnp.full((_L, 1), k, jnp.int32),
        dimension_numbers=lax.GatherDimensionNumbers(
            offset_dims=(), collapsed_slice_dims=(0,), start_index_map=(0,)),
        slice_sizes=(1,),
        mode=lax.GatherScatterMode.PROMISE_IN_BOUNDS)


def _fused_sc(tu, ti, users, items, ctx_flat, wts):
    mesh = plsc.VectorSubcoreMesh(core_axis_name="c", subcore_axis_name="s")

    @functools.partial(
        pl.kernel,
        mesh=mesh,
        compiler_params=pltpu.CompilerParams(needs_layout_passes=False),
        out_type=jax.ShapeDtypeStruct((_B,), jnp.float32),
        scratch_types=[
            pltpu.VMEM((_BPW,), jnp.int32),
            pltpu.VMEM((_BPW,), jnp.int32),
            pltpu.VMEM((_BPW,), jnp.int32),
            pltpu.VMEM((_BPW,), jnp.int32),
            [pltpu.VMEM((_CH,), jnp.int32) for _ in range(_NCHUNK)],
            [pltpu.VMEM((_CH,), jnp.int32) for _ in range(_NCHUNK)],
            pltpu.VMEM((_CH, 128), jnp.float32),
            pltpu.VMEM((_CH, 128), jnp.float32),
            pltpu.VMEM((_BPW * _ROWW,), jnp.float32),
            pltpu.VMEM((_BPW * _NCTX,), jnp.float32),
            pltpu.VMEM((_NWTS,), jnp.float32),
            pltpu.VMEM((_BPW,), jnp.float32),
            pltpu.SemaphoreType.DMA,
            pltpu.SemaphoreType.DMA,
        ],
    )
    def k(tu_h, ti_h, us_h, it_h, cx_h, wt_h, out_h,
          uidx, iidx, uoffs, ioffs, urows, irows, ubig, ibig,
          uic, cbuf, wbuf, obuf, usem, isem):
        wid = lax.axis_index("s") * _NC + lax.axis_index("c")
        base = wid * _BPW
        pltpu.sync_copy(us_h.at[pl.ds(base, _BPW)], uidx)
        pltpu.sync_copy(it_h.at[pl.ds(base, _BPW)], iidx)
        pltpu.sync_copy(cx_h.at[pl.ds(base * _NCTX, _BPW * _NCTX)], cbuf)
        pltpu.sync_copy(wt_h, wbuf)
        iota = lax.iota(jnp.int32, _L)

        for c in range(_NCHUNK):
            def rowoff(g, _, c=c):
                p = c * _CH + g * _L
                u = uidx[pl.ds(p, _L)]
                i = iidx[pl.ds(p, _L)]
                urows[c][pl.ds(g * _L, _L)] = u >> 3
                irows[c][pl.ds(g * _L, _L)] = i >> 3
                uoffs[pl.ds(p, _L)] = u & 7
                ioffs[pl.ds(p, _L)] = i & 7
                return _
            lax.fori_loop(0, _CH // _L, rowoff, None)

        for c in range(_NCHUNK):
            cu = pltpu.async_copy(tu_h.at[urows[c]], ubig, usem)
            ci = pltpu.async_copy(ti_h.at[irows[c]], ibig, isem)

            def extract(g, _, c=c, which=0):
                big, offs, col0 = ((ubig, uoffs, 0), (ibig, ioffs, _EMB))[which]
                rowv = g * _L + iota
                offv = offs[pl.ds(c * _CH + g * _L, _L)]
                colb = offv * _EMB
                flatb = (c * _CH + g * _L + iota) * _ROWW + col0
                for kk in range(_EMB):
                    vals = plsc.load_gather(big, [rowv, colb + kk])
                    plsc.store_scatter(uic, [flatb + kk], vals)
                return _

            cu.wait()
            lax.fori_loop(0, _CH // _L, functools.partial(extract, which=0),
                          None)
            ci.wait()
            lax.fori_loop(0, _CH // _L, functools.partial(extract, which=1),
                          None)

        def ctx_extract(g, _):
            rowv = g * _L + iota
            srcb = rowv * _NCTX
            dstb = rowv * _ROWW + 2 * _EMB
            for j in range(_NCTX):
                vals = plsc.load_gather(cbuf, [srcb + j])
                plsc.store_scatter(uic, [dstb + j], vals)
            return _
        lax.fori_loop(0, _BPW // _L, ctx_extract, None)

        w1a = [wbuf[pl.ds(r * _HID, _L)] for r in range(_ROWW)]
        w1b = [wbuf[pl.ds(r * _HID + _L, _L)] for r in range(_ROWW)]
        b1a = wbuf[pl.ds(_ROWW * _HID, _L)]
        b1b = wbuf[pl.ds(_ROWW * _HID + _L, _L)]
        w2a = wbuf[pl.ds(_ROWW * _HID + _HID, _L)]
        w2b = wbuf[pl.ds(_ROWW * _HID + _HID + _L, _L)]
        b2v = wbuf[pl.ds(_ROWW * _HID + 2 * _HID, _L)]
        lane15 = iota == (_L - 1)

        def mlp(s, _):
            b = s * _ROWW
            v0 = uic[pl.ds(b, _L)]
            v1 = uic[pl.ds(b + _L, _L)]
            v2 = uic[pl.ds(b + 24, _L)]
            acc0, acc1 = b1a, b1b
            for kk in range(_L):
                sp = _splat(v0, kk)
                acc0 = acc0 + sp * w1a[kk]
                acc1 = acc1 + sp * w1b[kk]
            for kk in range(_L):
                sp = _splat(v1, kk)
                acc0 = acc0 + sp * w1a[_EMB + kk]
                acc1 = acc1 + sp * w1b[_EMB + kk]
            for kk in range(8, _L):
                sp = _splat(v2, kk)
                acc0 = acc0 + sp * w1a[24 + kk]
                acc1 = acc1 + sp * w1b[24 + kk]
            h0 = jnp.maximum(acc0, 0.0)
            h1 = jnp.maximum(acc1, 0.0)
            t = h0 * w2a + h1 * w2b + b2v
            tc = jnp.cumsum(t)
            sv = lax.broadcast_in_dim(s, (_L,), ())
            plsc.store_scatter(obuf, [sv], tc, mask=lane15)
            return _
        lax.fori_loop(0, _BPW, mlp, None)

        pltpu.sync_copy(obuf, out_h.at[pl.ds(base, _BPW)])

    return k(tu, ti, users, items, ctx_flat, wts)


def kernel(users, items, context_features, user_table, item_table,
           ctx_W, ctx_b, W1, b1, W2, b2):
    tu = user_table.reshape(-1, 128)
    ti = item_table.reshape(-1, 128)
    cx = context_features.reshape(-1)
    w1c = W1[2 * _EMB:, :]
    w1f = jnp.concatenate([W1[:2 * _EMB, :], ctx_W @ w1c], axis=0)
    b1f = b1 + ctx_b @ w1c
    wts = jnp.concatenate([
        w1f.reshape(-1), b1f, W2[:, 0],
        jnp.pad(b2, (0, 15)),
    ])
    return _fused_sc(tu, ti, users.astype(jnp.int32), items.astype(jnp.int32),
                     cx, wts)
